# baseline (speedup 1.0000x reference)
import jax
import jax.numpy as jnp
from jax import lax
from jax.experimental import pallas as pl
from jax.experimental.pallas import tpu as pltpu
from jax.experimental.pallas import tpu_sc as plsc

NN = 10000
NPAD = 10240
EE = 320000
GG = 64
NS = 16
ET = EE // NS
KC = 128
NCHUNK = 158
ETP = NCHUNK * KC
NPT = NPAD // NS
NROW = NPAD // 16
NRT = NROW // NS
FQ = 32


def _lrelu_scalar(m):
    return jnp.where(m >= 0.0, m, 0.2 * m)


def _attention_scalars(h, asrc_ref, adst_ref, s_ref, t_ref, m_ref):
    s = jnp.dot(h, asrc_ref[...], preferred_element_type=jnp.float32)
    t = jnp.dot(h, adst_ref[...], preferred_element_type=jnp.float32)
    pad = jnp.zeros((NPAD - NN, 1), jnp.float32)
    s_ref[...] = jnp.concatenate([s, pad], axis=0)
    t_ref[...] = jnp.concatenate([t, pad], axis=0)
    m = _lrelu_scalar(jnp.max(s) + jnp.max(t))
    m_ref[...] = jnp.full((16, 1), m, jnp.float32)


def _tc_pre1(x_ref, w_ref, asrc_ref, adst_ref, hs_ref, s_ref, t_ref, m_ref):
    h = jnp.dot(x_ref[...], w_ref[...], preferred_element_type=jnp.float32)
    for q in range(4):
        hs_ref[q] = h[:, q * FQ:(q + 1) * FQ]
    _attention_scalars(h, asrc_ref, adst_ref, s_ref, t_ref, m_ref)


def _tc_mid(o_ref, den_ref, b1_ref, w2_ref, asrc_ref, adst_ref, hs_ref, s_ref,
            t_ref, m_ref):
    agg = o_ref[...][:NN] / (den_ref[...][:NN, :1] + 1e-16)
    h1 = jnp.maximum(agg + b1_ref[...], 0.0)
    h2 = jnp.dot(h1, w2_ref[...], preferred_element_type=jnp.float32)
    for q in range(2):
        hs_ref[q] = h2[:, q * FQ:(q + 1) * FQ]
    _attention_scalars(h2, asrc_ref, adst_ref, s_ref, t_ref, m_ref)


def _tc_post(o_ref, den_ref, b2_ref, wl_ref, bl_ref, batch_ref, wx1_ref, bx1_ref,
             wx2_ref, bx2_ref, we1_ref, be1_ref, we2_ref, be2_ref,
             xh_ref, eh_ref, z_ref, ge_ref):
    h = o_ref[...][:NN] / (den_ref[...][:NN, :1] + 1e-16) + b2_ref[...]
    z = jnp.dot(h, wl_ref[...], preferred_element_type=jnp.float32)
    z = z + bl_ref[...]
    z_ref[...] = z
    gid = lax.broadcasted_iota(jnp.int32, (GG, NN), 0)
    onehot = (batch_ref[...] == gid).astype(jnp.float32)
    sums = jnp.dot(onehot, z, preferred_element_type=jnp.float32)
    counts = jnp.sum(onehot, axis=1, keepdims=True)
    ge_ref[...] = sums / jnp.maximum(counts, 1.0)
    hx = jnp.maximum(
        jnp.dot(z, wx1_ref[...], preferred_element_type=jnp.float32)
        + bx1_ref[...], 0.0)
    xh_ref[...] = (jnp.dot(hx, wx2_ref[...],
                           preferred_element_type=jnp.float32) + bx2_ref[...])
    he = jnp.maximum(
        jnp.dot(z, we1_ref[...], preferred_element_type=jnp.float32)
        + be1_ref[...], 0.0)
    eh_ref[...] = (jnp.dot(he, we2_ref[...],
                           preferred_element_type=jnp.float32) + be2_ref[...])


def _make_sc_edge(npass):
    fv = FQ // 16

    def body(srcp_h, dstp_h, s_h, t_h, m_h, hs_h, out_h, den_h,
             sidx, didx, sloc, tloc, mloc, exb,
             rows0, rows1, rows2, exc0, exc1, exc2,
             gs0, gs1, gs2, ss0, ss1, ss2, es0, es1, es2,
             acc, den2):
        c = lax.axis_index("c")
        sid = lax.axis_index("s")

        pltpu.sync_copy(srcp_h.at[sid], sidx)
        pltpu.sync_copy(dstp_h.at[sid], didx)
        pltpu.sync_copy(s_h, sloc)
        pltpu.sync_copy(t_h, tloc)
        pltpu.sync_copy(m_h, mloc)

        zero16 = jnp.zeros((16,), jnp.float32)

        mv = mloc[...]

        def _pass_a(g, carry):
            ch = lax.shift_right_logical(g, 3)
            off = (g & 7) * 16
            sv = sidx[ch, pl.ds(off, 16)]
            dv = didx[ch, pl.ds(off, 16)]
            e = (plsc.load_gather(sloc, [lax.shift_right_logical(sv, 4),
                                         sv & 15])
                 + plsc.load_gather(tloc, [lax.shift_right_logical(dv, 4),
                                           dv & 15]))
            e = jnp.where(e >= 0.0, e, 0.2 * e)
            exb[ch, pl.ds(off, 16)] = jnp.exp(e - mv)
            return carry

        lax.fori_loop(0, ETP // 16, _pass_a, 0, unroll=2)

        e0 = jnp.where(lax.iota(jnp.int32, 16) == 0, 1.0, 0.0)

        bufs = (rows0, rows1, rows2)
        excs = (exc0, exc1, exc2)
        gsem = (gs0, gs1, gs2)
        ssem = (ss0, ss1, ss2)
        esem = (es0, es1, es2)
        LAST = NCHUNK - 1

        for p in range(npass):
            plane = c * npass + p

            def _zero_exc(i, carry):
                exc0[i, ...] = zero16
                return carry

            def _zero_rows(i, carry):
                for f in range(fv):
                    rows0[i, pl.ds(f * 16, 16)] = zero16
                return carry

            lax.fori_loop(0, KC, _zero_exc, 0)
            lax.fori_loop(0, KC, _zero_rows, 0)
            plsc.subcore_barrier()
            for j in range(NPT // KC):
                pltpu.sync_copy(rows0, acc.at[pl.ds(sid * NPT + j * KC, KC)])
                if p == 0:
                    pltpu.sync_copy(exc0,
                                    den2.at[pl.ds(sid * NPT + j * KC, KC)])
            plsc.subcore_barrier()

            def _scale(j, bi):
                buf = bufs[bi]
                excb = excs[bi]

                def _s4(t, carry):
                    for l in range(4):
                        i = t * 4 + l
                        ai = plsc.load_gather(
                            exb, [jnp.full((16,), j, jnp.int32),
                                  jnp.full((16,), i, jnp.int32)])
                        for f in range(fv):
                            buf[i, pl.ds(f * 16, 16)] = (
                                buf[i, pl.ds(f * 16, 16)] * ai)
                        if p == 0:
                            excb[i, ...] = ai * e0
                    return carry

                lax.fori_loop(0, KC // 4, _s4, 0, unroll=4)

            def _gather(j, bi):
                pltpu.async_copy(hs_h.at[plane].at[sidx.at[j]], bufs[bi],
                                 gsem[bi])

            def _wait_gather(j, bi):
                pltpu.make_async_copy(hs_h.at[plane].at[sidx.at[j]],
                                      bufs[bi], gsem[bi]).wait()

            def _scatter(j, bi):
                pltpu.async_copy(bufs[bi], acc.at[didx.at[j]], ssem[bi],
                                 add=True)
                if p == 0:
                    pltpu.async_copy(excs[bi], den2.at[didx.at[j]],
                                     esem[bi], add=True)

            def _wait_scatter(j, bi):
                pltpu.make_async_copy(bufs[bi], acc.at[didx.at[j]],
                                      ssem[bi]).wait()
                if p == 0:
                    pltpu.make_async_copy(excs[bi], den2.at[didx.at[j]],
                                          esem[bi]).wait()

            _gather(0, 0)
            _wait_gather(0, 0)
            _gather(1, 1)
            _scale(0, 0)
            _scatter(0, 0)
            _wait_gather(1, 1)
            _gather(2, 2)
            _scale(1, 1)
            _scatter(1, 1)

            def _main(t, carry):
                for i in range(3):
                    j = 2 + t * 3 + i
                    bi = (2 + i) % 3
                    bn = (bi + 1) % 3
                    _wait_gather(j, bi)
                    _wait_scatter(j - 2, bn)
                    _gather(jnp.minimum(j + 1, LAST), bn)
                    _scale(j, bi)
                    _scatter(j, bi)
                return carry

            lax.fori_loop(0, (NCHUNK - 2) // 3, _main, 0)

            _wait_gather(LAST, 2)
            _wait_scatter(NCHUNK - 2, 0)
            _wait_scatter(LAST, 1)

            plsc.subcore_barrier()
            pltpu.sync_copy(acc.at[pl.ds(sid * NPT, NPT)],
                            out_h.at[pl.ds(sid * NPT, NPT), plane])
            if p == 0:
                pltpu.sync_copy(den2.at[pl.ds(sid * NPT, NPT)],
                                den_h.at[pl.ds(sid * NPT, NPT)])

    mesh = plsc.VectorSubcoreMesh(core_axis_name="c", subcore_axis_name="s")
    return pl.kernel(
        body,
        out_type=[jax.ShapeDtypeStruct((NPAD, 2 * npass, FQ), jnp.float32),
                  jax.ShapeDtypeStruct((NPAD, 16), jnp.float32)],
        mesh=mesh,
        compiler_params=pltpu.CompilerParams(needs_layout_passes=False,
                                             use_tc_tiling_on_sc=False),
        scratch_types=[
            pltpu.VMEM((NCHUNK, KC), jnp.int32),
            pltpu.VMEM((NCHUNK, KC), jnp.int32),
            pltpu.VMEM((NROW, 16), jnp.float32),
            pltpu.VMEM((NROW, 16), jnp.float32),
            pltpu.VMEM((16,), jnp.float32),
            pltpu.VMEM((NCHUNK, KC), jnp.float32),
            pltpu.VMEM((KC, FQ), jnp.float32),
            pltpu.VMEM((KC, FQ), jnp.float32),
            pltpu.VMEM((KC, FQ), jnp.float32),
            pltpu.VMEM((KC, 16), jnp.float32),
            pltpu.VMEM((KC, 16), jnp.float32),
            pltpu.VMEM((KC, 16), jnp.float32),
            pltpu.SemaphoreType.DMA,
            pltpu.SemaphoreType.DMA,
            pltpu.SemaphoreType.DMA,
            pltpu.SemaphoreType.DMA,
            pltpu.SemaphoreType.DMA,
            pltpu.SemaphoreType.DMA,
            pltpu.SemaphoreType.DMA,
            pltpu.SemaphoreType.DMA,
            pltpu.SemaphoreType.DMA,
            pltpu.VMEM_SHARED((NPAD, FQ), jnp.float32),
            pltpu.VMEM_SHARED((NPAD, 16), jnp.float32),
        ],
    )


_sc_edge_l1 = _make_sc_edge(2)
_sc_edge_l2 = _make_sc_edge(1)


def _pad_edges(edge_index):
    src = edge_index[0].astype(jnp.int32).reshape(NS, ET)
    dst = edge_index[1].astype(jnp.int32).reshape(NS, ET)
    srcp = jnp.concatenate(
        [src, jnp.zeros((NS, ETP - ET), jnp.int32)], axis=1)
    dstp = jnp.concatenate(
        [dst, jnp.full((NS, ETP - ET), NN, jnp.int32)], axis=1)
    return (srcp.reshape(NS, NCHUNK, KC), dstp.reshape(NS, NCHUNK, KC))


def kernel(x, edge_index, batch, edge_attr, W1, att_src1, att_dst1, b1,
           W2, att_src2, att_dst2, b2, Wl, bl, Wx1, bx1, Wx2, bx2,
           We1, be1, We2, be2):
    del edge_attr
    f32 = jnp.float32
    srcp, dstp = _pad_edges(edge_index)

    tc_params = pltpu.CompilerParams(vmem_limit_bytes=100 * 1024 * 1024)
    hs1, s1, t1, m1 = pl.pallas_call(
        _tc_pre1,
        compiler_params=tc_params,
        out_shape=[jax.ShapeDtypeStruct((4, NN, FQ), f32),
                   jax.ShapeDtypeStruct((NPAD, 1), f32),
                   jax.ShapeDtypeStruct((NPAD, 1), f32),
                   jax.ShapeDtypeStruct((16, 1), f32)],
    )(x, W1, att_src1.reshape(-1, 1), att_dst1.reshape(-1, 1))

    o1, d1 = _sc_edge_l1(srcp, dstp, s1.reshape(NROW, 16),
                         t1.reshape(NROW, 16), m1.reshape(16), hs1)

    hs2, s2, t2, m2 = pl.pallas_call(
        _tc_mid,
        compiler_params=tc_params,
        out_shape=[jax.ShapeDtypeStruct((2, NN, FQ), f32),
                   jax.ShapeDtypeStruct((NPAD, 1), f32),
                   jax.ShapeDtypeStruct((NPAD, 1), f32),
                   jax.ShapeDtypeStruct((16, 1), f32)],
    )(o1.reshape(NPAD, 128), d1, b1.reshape(1, -1), W2, att_src2.reshape(-1, 1),
      att_dst2.reshape(-1, 1))

    o2, d2 = _sc_edge_l2(srcp, dstp, s2.reshape(NROW, 16),
                         t2.reshape(NROW, 16), m2.reshape(16), hs2)

    we2p = jnp.pad(We2, ((0, 0), (0, 128 - We2.shape[1])))
    be2p = jnp.pad(be2, (0, 128 - be2.shape[0])).reshape(1, -1)
    xh, eh, z, ge = pl.pallas_call(
        _tc_post,
        compiler_params=tc_params,
        out_shape=[jax.ShapeDtypeStruct((NN, 128), f32),
                   jax.ShapeDtypeStruct((NN, 128), f32),
                   jax.ShapeDtypeStruct((NN, 64), f32),
                   jax.ShapeDtypeStruct((GG, 64), f32)],
    )(o2.reshape(NPAD, 64), d2, b2.reshape(1, -1), Wl, bl.reshape(1, -1),
      batch.astype(jnp.int32).reshape(1, NN), Wx1, bx1.reshape(1, -1),
      Wx2, bx2.reshape(1, -1), We1, be1.reshape(1, -1), we2p, be2p)

    return (xh, eh[:, :3], z, ge)


# speedup vs baseline: 1.1788x; 1.1788x over previous
import jax
import jax.numpy as jnp
from jax import lax
from jax.experimental import pallas as pl
from jax.experimental.pallas import tpu as pltpu
from jax.experimental.pallas import tpu_sc as plsc

NN = 10000
NPAD = 10240
EE = 320000
GG = 64
NS = 16
ET = EE // NS
KC = 128
NCHUNK = 158
ETP = NCHUNK * KC
NPT = NPAD // NS
NROW = NPAD // 16
NRT = NROW // NS
FQ = 32


def _lrelu_scalar(m):
    return jnp.where(m >= 0.0, m, 0.2 * m)


def _attention_scalars(h, asrc_ref, adst_ref, s_ref, t_ref, m_ref):
    s = jnp.dot(h, asrc_ref[...], preferred_element_type=jnp.float32)
    t = jnp.dot(h, adst_ref[...], preferred_element_type=jnp.float32)
    pad = jnp.zeros((NPAD - NN, 1), jnp.float32)
    s_ref[...] = jnp.concatenate([s, pad], axis=0)
    t_ref[...] = jnp.concatenate([t, pad], axis=0)
    m = _lrelu_scalar(jnp.max(s) + jnp.max(t))
    m_ref[...] = jnp.full((16, 1), m, jnp.float32)


def _tc_pre1(x_ref, w_ref, asrc_ref, adst_ref, hs_ref, s_ref, t_ref, m_ref):
    h = jnp.dot(x_ref[...], w_ref[...], preferred_element_type=jnp.float32)
    for q in range(4):
        hs_ref[q] = h[:, q * FQ:(q + 1) * FQ]
    _attention_scalars(h, asrc_ref, adst_ref, s_ref, t_ref, m_ref)


def _tc_mid(o_ref, den_ref, b1_ref, w2_ref, asrc_ref, adst_ref, hs_ref, s_ref,
            t_ref, m_ref):
    agg = o_ref[...][:NN] / (den_ref[...][:NN, :1] + 1e-16)
    h1 = jnp.maximum(agg + b1_ref[...], 0.0)
    h2 = jnp.dot(h1, w2_ref[...], preferred_element_type=jnp.float32)
    for q in range(2):
        hs_ref[q] = h2[:, q * FQ:(q + 1) * FQ]
    _attention_scalars(h2, asrc_ref, adst_ref, s_ref, t_ref, m_ref)


def _tc_post(o_ref, den_ref, b2_ref, wl_ref, bl_ref, batch_ref, wx1_ref, bx1_ref,
             wx2_ref, bx2_ref, we1_ref, be1_ref, we2_ref, be2_ref,
             xh_ref, eh_ref, z_ref, ge_ref):
    h = o_ref[...][:NN] / (den_ref[...][:NN, :1] + 1e-16) + b2_ref[...]
    z = jnp.dot(h, wl_ref[...], preferred_element_type=jnp.float32)
    z = z + bl_ref[...]
    z_ref[...] = z
    gid = lax.broadcasted_iota(jnp.int32, (GG, NN), 0)
    onehot = (batch_ref[...] == gid).astype(jnp.float32)
    sums = jnp.dot(onehot, z, preferred_element_type=jnp.float32)
    counts = jnp.sum(onehot, axis=1, keepdims=True)
    ge_ref[...] = sums / jnp.maximum(counts, 1.0)
    hx = jnp.maximum(
        jnp.dot(z, wx1_ref[...], preferred_element_type=jnp.float32)
        + bx1_ref[...], 0.0)
    xh_ref[...] = (jnp.dot(hx, wx2_ref[...],
                           preferred_element_type=jnp.float32) + bx2_ref[...])
    he = jnp.maximum(
        jnp.dot(z, we1_ref[...], preferred_element_type=jnp.float32)
        + be1_ref[...], 0.0)
    eh_ref[...] = (jnp.dot(he, we2_ref[...],
                           preferred_element_type=jnp.float32) + be2_ref[...])


def _make_sc_edge(npass):
    fv = FQ // 16

    def body(srcp_h, dstp_h, s_h, t_h, m_h, hs_h, out_h, den_h,
             sidx, didx, sloc, tloc, mloc, exb,
             rows0, rows1, rows2, exc0, exc1, exc2,
             gs0, gs1, gs2, ss0, ss1, ss2, es0, es1, es2,
             acc, den2):
        c = lax.axis_index("c")
        sid = lax.axis_index("s")

        pltpu.sync_copy(srcp_h.at[sid], sidx)
        pltpu.sync_copy(dstp_h.at[sid], didx)
        pltpu.sync_copy(s_h, sloc)
        pltpu.sync_copy(t_h, tloc)
        pltpu.sync_copy(m_h, mloc)

        zero16 = jnp.zeros((16,), jnp.float32)

        mv = mloc[...]

        def _pass_a(g, carry):
            ch = lax.shift_right_logical(g, 3)
            off = (g & 7) * 16
            sv = sidx[ch, pl.ds(off, 16)]
            dv = didx[ch, pl.ds(off, 16)]
            e = (plsc.load_gather(sloc, [lax.shift_right_logical(sv, 4),
                                         sv & 15])
                 + plsc.load_gather(tloc, [lax.shift_right_logical(dv, 4),
                                           dv & 15]))
            e = jnp.where(e >= 0.0, e, 0.2 * e)
            exb[ch, pl.ds(off, 16)] = jnp.exp(e - mv)
            return carry

        lax.fori_loop(0, ETP // 16, _pass_a, 0)

        e0 = jnp.where(lax.iota(jnp.int32, 16) == 0, 1.0, 0.0)

        bufs = (rows0, rows1, rows2)
        excs = (exc0, exc1, exc2)
        gsem = (gs0, gs1, gs2)
        ssem = (ss0, ss1, ss2)
        esem = (es0, es1, es2)
        LAST = NCHUNK - 1

        for p in range(npass):
            plane = c * npass + p

            def _zero_exc(i, carry):
                exc0[i, ...] = zero16
                return carry

            def _zero_rows(i, carry):
                for f in range(fv):
                    rows0[i, pl.ds(f * 16, 16)] = zero16
                return carry

            lax.fori_loop(0, KC, _zero_exc, 0)
            lax.fori_loop(0, KC, _zero_rows, 0)
            plsc.subcore_barrier()
            for j in range(NPT // KC):
                pltpu.sync_copy(rows0, acc.at[pl.ds(sid * NPT + j * KC, KC)])
                if p == 0:
                    pltpu.sync_copy(exc0,
                                    den2.at[pl.ds(sid * NPT + j * KC, KC)])
            plsc.subcore_barrier()

            def _scale(j, bi):
                buf = bufs[bi]
                excb = excs[bi]

                def _s16(t, carry):
                    base = t * 16
                    av = exb[j, pl.ds(base, 16)]
                    for l in range(16):
                        i = base + l
                        ai = av[l]
                        for f in range(fv):
                            buf[i, pl.ds(f * 16, 16)] = (
                                buf[i, pl.ds(f * 16, 16)] * ai)
                        if p == 0:
                            excb[i, ...] = ai * e0
                    return carry

                lax.fori_loop(0, KC // 16, _s16, 0)

            def _gather(j, bi):
                pltpu.async_copy(hs_h.at[plane].at[sidx.at[j]], bufs[bi],
                                 gsem[bi])

            def _wait_gather(j, bi):
                pltpu.make_async_copy(hs_h.at[plane].at[sidx.at[j]],
                                      bufs[bi], gsem[bi]).wait()

            def _scatter(j, bi):
                pltpu.async_copy(bufs[bi], acc.at[didx.at[j]], ssem[bi],
                                 add=True)
                if p == 0:
                    pltpu.async_copy(excs[bi], den2.at[didx.at[j]],
                                     esem[bi], add=True)

            def _wait_scatter(j, bi):
                pltpu.make_async_copy(bufs[bi], acc.at[didx.at[j]],
                                      ssem[bi]).wait()
                if p == 0:
                    pltpu.make_async_copy(excs[bi], den2.at[didx.at[j]],
                                          esem[bi]).wait()

            _gather(0, 0)
            _wait_gather(0, 0)
            _gather(1, 1)
            _scale(0, 0)
            _scatter(0, 0)
            _wait_gather(1, 1)
            _gather(2, 2)
            _scale(1, 1)
            _scatter(1, 1)

            def _main(t, carry):
                for i in range(3):
                    j = 2 + t * 3 + i
                    bi = (2 + i) % 3
                    bn = (bi + 1) % 3
                    _wait_gather(j, bi)
                    _wait_scatter(j - 2, bn)
                    _gather(jnp.minimum(j + 1, LAST), bn)
                    _scale(j, bi)
                    _scatter(j, bi)
                return carry

            lax.fori_loop(0, (NCHUNK - 2) // 3, _main, 0)

            _wait_gather(LAST, 2)
            _wait_scatter(NCHUNK - 2, 0)
            _wait_scatter(LAST, 1)

            plsc.subcore_barrier()
            pltpu.sync_copy(acc.at[pl.ds(sid * NPT, NPT)],
                            out_h.at[pl.ds(sid * NPT, NPT), plane])
            if p == 0:
                pltpu.sync_copy(den2.at[pl.ds(sid * NPT, NPT)],
                                den_h.at[pl.ds(sid * NPT, NPT)])

    mesh = plsc.VectorSubcoreMesh(core_axis_name="c", subcore_axis_name="s")
    return pl.kernel(
        body,
        out_type=[jax.ShapeDtypeStruct((NPAD, 2 * npass, FQ), jnp.float32),
                  jax.ShapeDtypeStruct((NPAD, 16), jnp.float32)],
        mesh=mesh,
        compiler_params=pltpu.CompilerParams(needs_layout_passes=False,
                                             use_tc_tiling_on_sc=False),
        scratch_types=[
            pltpu.VMEM((NCHUNK, KC), jnp.int32),
            pltpu.VMEM((NCHUNK, KC), jnp.int32),
            pltpu.VMEM((NROW, 16), jnp.float32),
            pltpu.VMEM((NROW, 16), jnp.float32),
            pltpu.VMEM((16,), jnp.float32),
            pltpu.VMEM((NCHUNK, KC), jnp.float32),
            pltpu.VMEM((KC, FQ), jnp.float32),
            pltpu.VMEM((KC, FQ), jnp.float32),
            pltpu.VMEM((KC, FQ), jnp.float32),
            pltpu.VMEM((KC, 16), jnp.float32),
            pltpu.VMEM((KC, 16), jnp.float32),
            pltpu.VMEM((KC, 16), jnp.float32),
            pltpu.SemaphoreType.DMA,
            pltpu.SemaphoreType.DMA,
            pltpu.SemaphoreType.DMA,
            pltpu.SemaphoreType.DMA,
            pltpu.SemaphoreType.DMA,
            pltpu.SemaphoreType.DMA,
            pltpu.SemaphoreType.DMA,
            pltpu.SemaphoreType.DMA,
            pltpu.SemaphoreType.DMA,
            pltpu.VMEM_SHARED((NPAD, FQ), jnp.float32),
            pltpu.VMEM_SHARED((NPAD, 16), jnp.float32),
        ],
    )


_sc_edge_l1 = _make_sc_edge(2)
_sc_edge_l2 = _make_sc_edge(1)


def _pad_edges(edge_index):
    src = edge_index[0].astype(jnp.int32).reshape(NS, ET)
    dst = edge_index[1].astype(jnp.int32).reshape(NS, ET)
    srcp = jnp.concatenate(
        [src, jnp.zeros((NS, ETP - ET), jnp.int32)], axis=1)
    dstp = jnp.concatenate(
        [dst, jnp.full((NS, ETP - ET), NN, jnp.int32)], axis=1)
    return (srcp.reshape(NS, NCHUNK, KC), dstp.reshape(NS, NCHUNK, KC))


def kernel(x, edge_index, batch, edge_attr, W1, att_src1, att_dst1, b1,
           W2, att_src2, att_dst2, b2, Wl, bl, Wx1, bx1, Wx2, bx2,
           We1, be1, We2, be2):
    del edge_attr
    f32 = jnp.float32
    srcp, dstp = _pad_edges(edge_index)

    tc_params = pltpu.CompilerParams(vmem_limit_bytes=100 * 1024 * 1024)
    hs1, s1, t1, m1 = pl.pallas_call(
        _tc_pre1,
        compiler_params=tc_params,
        out_shape=[jax.ShapeDtypeStruct((4, NN, FQ), f32),
                   jax.ShapeDtypeStruct((NPAD, 1), f32),
                   jax.ShapeDtypeStruct((NPAD, 1), f32),
                   jax.ShapeDtypeStruct((16, 1), f32)],
    )(x, W1, att_src1.reshape(-1, 1), att_dst1.reshape(-1, 1))

    o1, d1 = _sc_edge_l1(srcp, dstp, s1.reshape(NROW, 16),
                         t1.reshape(NROW, 16), m1.reshape(16), hs1)

    hs2, s2, t2, m2 = pl.pallas_call(
        _tc_mid,
        compiler_params=tc_params,
        out_shape=[jax.ShapeDtypeStruct((2, NN, FQ), f32),
                   jax.ShapeDtypeStruct((NPAD, 1), f32),
                   jax.ShapeDtypeStruct((NPAD, 1), f32),
                   jax.ShapeDtypeStruct((16, 1), f32)],
    )(o1.reshape(NPAD, 128), d1, b1.reshape(1, -1), W2, att_src2.reshape(-1, 1),
      att_dst2.reshape(-1, 1))

    o2, d2 = _sc_edge_l2(srcp, dstp, s2.reshape(NROW, 16),
                         t2.reshape(NROW, 16), m2.reshape(16), hs2)

    we2p = jnp.pad(We2, ((0, 0), (0, 128 - We2.shape[1])))
    be2p = jnp.pad(be2, (0, 128 - be2.shape[0])).reshape(1, -1)
    xh, eh, z, ge = pl.pallas_call(
        _tc_post,
        compiler_params=tc_params,
        out_shape=[jax.ShapeDtypeStruct((NN, 128), f32),
                   jax.ShapeDtypeStruct((NN, 128), f32),
                   jax.ShapeDtypeStruct((NN, 64), f32),
                   jax.ShapeDtypeStruct((GG, 64), f32)],
    )(o2.reshape(NPAD, 64), d2, b2.reshape(1, -1), Wl, bl.reshape(1, -1),
      batch.astype(jnp.int32).reshape(1, NN), Wx1, bx1.reshape(1, -1),
      Wx2, bx2.reshape(1, -1), We1, be1.reshape(1, -1), we2p, be2p)

    return (xh, eh[:, :3], z, ge)
